# baseline (speedup 1.0000x reference)
import functools

import jax
import jax.numpy as jnp
from jax import lax
from jax.experimental import pallas as pl
from jax.experimental.pallas import tpu as pltpu
from jax.experimental.pallas import tpu_sc as plsc

_BATCH = 16384
_DIM = 64
_MARGIN = 2.0

_NC = 2
_NS = 16
_L = 16
_NW = _NC * _NS
_BPW = _BATCH // _NW
_CH = 16
_NCHUNK = _BPW // _CH
_W = _CH * _DIM


def _sqrt16(x):
    i = plsc.bitcast(x, jnp.int32)
    y = plsc.bitcast(jnp.int32(0x5F3759DF) - (i >> 1), jnp.float32)
    for _ in range(3):
        y = y * (1.5 - 0.5 * x * y * y)
    return x * y


def _tec_body(hs_hbm, ts_hbm, rs_hbm, ent_hbm, rel_hbm, nv_hbm, out_hbm,
              hv, tv, rv, hidx, tidx, ridx, h_flat, t_flat, r_flat, n_flat,
              u_buf, out_v, sem_a, sem_b):
    wid = lax.axis_index("s") * _NC + lax.axis_index("c")
    base = wid * _BPW
    lane = lax.broadcasted_iota(jnp.int32, (_L,), 0)

    pltpu.sync_copy(hs_hbm.at[pl.ds(base, _BPW)], hv)
    pltpu.sync_copy(ts_hbm.at[pl.ds(base, _BPW)], tv)
    pltpu.sync_copy(rs_hbm.at[pl.ds(base, _BPW)], rv)

    def fire(c, slot, sem):
        hb = hv[pl.ds(c * _CH, _CH)] * _DIM
        tb = tv[pl.ds(c * _CH, _CH)] * _DIM
        rb = rv[pl.ds(c * _CH, _CH)] * _DIM
        for w in range(_DIM):
            sl = pl.ds(w * _L, _L)
            hidx[slot, sl] = hb + w
            tidx[slot, sl] = tb + w
            ridx[slot, sl] = rb + w
        pltpu.async_copy(ent_hbm.at[hidx.at[slot]], h_flat.at[slot], sem)
        pltpu.async_copy(ent_hbm.at[tidx.at[slot]], t_flat.at[slot], sem)
        pltpu.async_copy(rel_hbm.at[ridx.at[slot]], r_flat.at[slot], sem)
        pltpu.async_copy(nv_hbm.at[ridx.at[slot]], n_flat.at[slot], sem)

    def drain(slot, sem):
        pltpu.make_async_copy(
            ent_hbm.at[pl.ds(0, _W)], h_flat.at[slot], sem).wait()
        pltpu.make_async_copy(
            ent_hbm.at[pl.ds(0, _W)], t_flat.at[slot], sem).wait()
        pltpu.make_async_copy(
            rel_hbm.at[pl.ds(0, _W)], r_flat.at[slot], sem).wait()
        pltpu.make_async_copy(
            nv_hbm.at[pl.ds(0, _W)], n_flat.at[slot], sem).wait()

    def compute(c, slot):
        bdot = jnp.zeros((_L,), jnp.float32)
        for w in range(_DIM):
            sl = pl.ds(w * _L, _L)
            u = h_flat[slot, sl] - t_flat[slot, sl]
            bdot = bdot + u * n_flat[slot, sl]
            u_buf[sl] = u
        acc = jnp.zeros((_L,), jnp.float32)
        for w in range(_DIM):
            sl = pl.ds(w * _L, _L)
            v = u_buf[sl] + r_flat[slot, sl] - bdot * n_flat[slot, sl]
            acc = acc + v * v
        out_v[pl.ds(c * _CH, _CH)] = _MARGIN - _sqrt16(acc)

    fire(0, 0, sem_a)

    def body2(m, carry):
        c0 = 2 * m
        c1 = c0 + 1
        fire(c1, 1, sem_b)
        drain(0, sem_a)
        compute(c0, 0)
        pl.when(c1 + 1 < _NCHUNK)(lambda: fire(c1 + 1, 0, sem_a))
        drain(1, sem_b)
        compute(c1, 1)
        return carry

    lax.fori_loop(0, _NCHUNK // 2, body2, 0)
    pltpu.sync_copy(out_v, out_hbm.at[pl.ds(base, _BPW)])


_mesh = plsc.VectorSubcoreMesh(core_axis_name="c", subcore_axis_name="s")

_sc_call = functools.partial(
    pl.kernel,
    mesh=_mesh,
    compiler_params=pltpu.CompilerParams(
        needs_layout_passes=False, use_tc_tiling_on_sc=False),
    out_type=jax.ShapeDtypeStruct((_BATCH,), jnp.float32),
    scratch_types=[
        pltpu.VMEM((_BPW,), jnp.int32),
        pltpu.VMEM((_BPW,), jnp.int32),
        pltpu.VMEM((_BPW,), jnp.int32),
        pltpu.VMEM((2, _W), jnp.int32),
        pltpu.VMEM((2, _W), jnp.int32),
        pltpu.VMEM((2, _W), jnp.int32),
        pltpu.VMEM((2, _W), jnp.float32),
        pltpu.VMEM((2, _W), jnp.float32),
        pltpu.VMEM((2, _W), jnp.float32),
        pltpu.VMEM((2, _W), jnp.float32),
        pltpu.VMEM((_W,), jnp.float32),
        pltpu.VMEM((_BPW,), jnp.float32),
        pltpu.SemaphoreType.DMA,
        pltpu.SemaphoreType.DMA,
    ],
)(_tec_body)


@jax.jit
def kernel(hs, rs, ts, ent_embs, rel_embs, norm_vector):
    scores = _sc_call(hs.astype(jnp.int32), ts.astype(jnp.int32),
                      rs.astype(jnp.int32), ent_embs.reshape(-1),
                      rel_embs.reshape(-1), norm_vector.reshape(-1))
    return scores.reshape(_BATCH, 1)


# speedup vs baseline: 2.0954x; 2.0954x over previous
import functools

import jax
import jax.numpy as jnp
from jax import lax
from jax.experimental import pallas as pl
from jax.experimental.pallas import tpu as pltpu
from jax.experimental.pallas import tpu_sc as plsc

_BATCH = 16384
_DIM = 64
_MARGIN = 2.0

_NC = 2
_NS = 16
_L = 16
_NW = _NC * _NS
_BPW = _BATCH // _NW
_CH = 16
_NCHUNK = _BPW // _CH
_NSEM = 4


def _sqrt16(x):
    i = plsc.bitcast(x, jnp.int32)
    y = plsc.bitcast(jnp.int32(0x5F3759DF) - (i >> 1), jnp.float32)
    for _ in range(3):
        y = y * (1.5 - 0.5 * x * y * y)
    return x * y


def _rsum(v):
    return lax.reduce_sum_p.bind(v, axes=(0,))


def _tec_body(hs_hbm, ts_hbm, rs_hbm, ent_hbm, rn_hbm, out_hbm,
              hv, tv, rv, h_rows, t_rows, rn_rows, out_v, *sems):
    wid = lax.axis_index("s") * _NC + lax.axis_index("c")
    base = wid * _BPW
    lane = lax.broadcasted_iota(jnp.int32, (_L,), 0)

    pltpu.sync_copy(hs_hbm.at[pl.ds(base, _BPW)], hv)
    pltpu.sync_copy(ts_hbm.at[pl.ds(base, _BPW)], tv)
    pltpu.sync_copy(rs_hbm.at[pl.ds(base, _BPW)], rv)

    def fire(c, slot):
        ss = sems[slot * _NSEM:(slot + 1) * _NSEM]
        rn_sem = sems[2 * _NSEM + slot]
        hvec = hv[pl.ds(c * _CH, _CH)]
        tvec = tv[pl.ds(c * _CH, _CH)]
        rvec = rv[pl.ds(c * _CH, _CH)]
        for i in range(_CH):
            pltpu.async_copy(
                ent_hbm.at[hvec[i]], h_rows.at[slot, i], ss[i % _NSEM])
            pltpu.async_copy(
                ent_hbm.at[tvec[i]], t_rows.at[slot, i], ss[i % _NSEM])
        pltpu.async_copy(rn_hbm.at[rvec], rn_rows.at[slot], rn_sem)

    def drain(slot):
        ss = sems[slot * _NSEM:(slot + 1) * _NSEM]
        rn_sem = sems[2 * _NSEM + slot]
        per_sem = 2 * _CH // _NSEM
        for q in range(_NSEM):
            pltpu.make_async_copy(
                ent_hbm.at[pl.ds(0, per_sem)],
                h_rows.at[slot, pl.ds(0, per_sem)], ss[q]).wait()
        pltpu.make_async_copy(
            rn_hbm.at[pl.ds(0, _CH)], rn_rows.at[slot], rn_sem).wait()

    def compute(c, slot):
        total_vec = jnp.zeros((_L,), jnp.float32)
        for i in range(_CH):
            us = []
            ns = []
            pb = jnp.zeros((_L,), jnp.float32)
            for k in range(_DIM // _L):
                sl = pl.ds(k * _L, _L)
                hk = h_rows[slot, i, sl]
                tk = t_rows[slot, i, sl]
                nk = rn_rows[slot, i, pl.ds(_DIM + k * _L, _L)]
                u = hk - tk
                pb = pb + u * nk
                us.append(u)
                ns.append(nk)
            b = _rsum(pb)
            pv = jnp.zeros((_L,), jnp.float32)
            for k in range(_DIM // _L):
                rk = rn_rows[slot, i, pl.ds(k * _L, _L)]
                v = us[k] + rk - b * ns[k]
                pv = pv + v * v
            tot = _rsum(pv)
            total_vec = jnp.where(lane == i, tot, total_vec)
        out_v[pl.ds(c * _CH, _CH)] = _MARGIN - _sqrt16(total_vec)

    fire(0, 0)

    def body2(m, carry):
        c0 = 2 * m
        c1 = c0 + 1
        fire(c1, 1)
        drain(0)
        compute(c0, 0)
        pl.when(c1 + 1 < _NCHUNK)(lambda: fire(c1 + 1, 0))
        drain(1)
        compute(c1, 1)
        return carry

    lax.fori_loop(0, _NCHUNK // 2, body2, 0)
    pltpu.sync_copy(out_v, out_hbm.at[pl.ds(base, _BPW)])


_mesh = plsc.VectorSubcoreMesh(core_axis_name="c", subcore_axis_name="s")

_sc_call = functools.partial(
    pl.kernel,
    mesh=_mesh,
    compiler_params=pltpu.CompilerParams(
        needs_layout_passes=False, use_tc_tiling_on_sc=True),
    out_type=jax.ShapeDtypeStruct((_BATCH,), jnp.float32),
    scratch_types=[
        pltpu.VMEM((_BPW,), jnp.int32),
        pltpu.VMEM((_BPW,), jnp.int32),
        pltpu.VMEM((_BPW,), jnp.int32),
        pltpu.VMEM((2, _CH, _DIM), jnp.float32),
        pltpu.VMEM((2, _CH, _DIM), jnp.float32),
        pltpu.VMEM((2, _CH, 2 * _DIM), jnp.float32),
        pltpu.VMEM((_BPW,), jnp.float32),
    ] + [pltpu.SemaphoreType.DMA] * (2 * _NSEM + 2),
)(_tec_body)


@jax.jit
def kernel(hs, rs, ts, ent_embs, rel_embs, norm_vector):
    rn = jnp.concatenate([rel_embs, norm_vector], axis=1)
    scores = _sc_call(hs.astype(jnp.int32), ts.astype(jnp.int32),
                      rs.astype(jnp.int32), ent_embs, rn)
    return scores.reshape(_BATCH, 1)
